# baseline (speedup 1.0000x reference)
import functools

import numpy as np
import jax
import jax.numpy as jnp
from jax.experimental import pallas as pl
from jax.experimental.pallas import tpu as pltpu

_MODULE_NUM = 8
_HALF = 512
_BLOCK = 2 * _HALF
_GROUP = 8
_NEG = -1e30


def _seg_butterfly(x, combine):
    lanes = x.shape[1]
    lane = jax.lax.broadcasted_iota(jnp.int32, x.shape, 1)
    for k in (1, 2, 4):
        fwd = jnp.roll(x, -k, axis=1)
        bwd = jnp.roll(x, k, axis=1)
        partner = jnp.where((lane & k) == 0, fwd, bwd)
        x = combine(x, partner)
    return x


def _routing_weights64(g64):
    i32 = jnp.int32
    lane = jax.lax.broadcasted_iota(i32, g64.shape, 1)
    m1 = _seg_butterfly(g64, jnp.maximum)
    i1 = _seg_butterfly(jnp.where(g64 >= m1, lane, 64), jnp.minimum)
    first1 = lane == i1
    masked = jnp.where(first1, _NEG, g64)
    m2 = _seg_butterfly(masked, jnp.maximum)
    i2 = _seg_butterfly(jnp.where(masked >= m2, lane, 64), jnp.minimum)
    first2 = lane == i2
    e2 = jnp.exp(m2 - m1)
    w1 = 1.0 / (1.0 + e2)
    zero = jnp.zeros_like(g64)
    return jnp.where(first1, w1, zero) + jnp.where(first2, 1.0 - w1, zero)


def _fused_body(mx_ref, gx_ref, wg0_ref, bg0_ref, wg1_ref, bg1_ref,
                wm_ref, bm_ref, out_ref):
    f32 = jnp.float32
    bf16 = jnp.bfloat16
    g1 = jnp.dot(gx_ref[...], wg0_ref[...], preferred_element_type=f32)
    g1 = jnp.maximum(g1 + bg0_ref[...], 0.0)
    g64 = jnp.dot(g1.astype(bf16), wg1_ref[...],
                  preferred_element_type=f32) + bg1_ref[...]
    wd = _routing_weights64(g64)

    _CH = 128
    rows = [slice(0, _HALF), slice(_HALF, _BLOCK)]
    h_dim = wm_ref.shape[2]
    for h in range(2):
        r = rows[h]
        a = jnp.dot(mx_ref[r, :], wm_ref[0], preferred_element_type=f32)
        out = jnp.maximum(a + bm_ref[0:1, :], 0.0)
        outs = [out.astype(bf16)]
        for j in range(1, _MODULE_NUM):
            c0 = _GROUP * (j - 1)
            wcols = [wd[r, c0 + i:c0 + i + 1] for i in range(j)]
            chunks = []
            for s0 in range(0, h_dim, _CH):
                s = slice(s0, s0 + _CH)
                accc = wcols[0] * outs[0][:, s]
                for i in range(1, j):
                    accc = accc + wcols[i] * outs[i][:, s]
                chunks.append(accc)
            fc_in = jnp.concatenate(chunks, axis=1)
            fc = jnp.dot(fc_in.astype(bf16), wm_ref[j],
                         preferred_element_type=f32)
            out = jnp.maximum(fc + bm_ref[j:j + 1, :], 0.0) + fc_in
            outs.append(out.astype(bf16))
        c0 = _GROUP * (_MODULE_NUM - 1)
        wcols = [wd[r, c0 + i:c0 + i + 1] for i in range(_MODULE_NUM)]
        for s0 in range(0, h_dim, _CH):
            s = slice(s0, s0 + _CH)
            accc = wcols[0] * outs[0][:, s]
            for i in range(1, _MODULE_NUM):
                accc = accc + wcols[i] * outs[i][:, s]
            out_ref[r, s] = accc


@functools.partial(jax.jit, static_argnames=("interpret",))
def _run(mx, gx, wg0, bg0, wg1, bg1, wm, bm, interpret=False):
    n, d_in = mx.shape
    h = wm.shape[2]
    gin = gx.shape[1]
    ghid = wg0.shape[1]
    gout = wg1.shape[1]
    grid = (n // _BLOCK,)
    full = lambda *s: pl.BlockSpec(s, lambda i: (0,) * len(s))
    return pl.pallas_call(
        _fused_body,
        grid=grid,
        in_specs=[
            pl.BlockSpec((_BLOCK, d_in), lambda i: (i, 0)),
            pl.BlockSpec((_BLOCK, gin), lambda i: (i, 0)),
            full(gin, ghid),
            full(1, ghid),
            full(ghid, gout),
            full(1, gout),
            full(_MODULE_NUM, d_in, h),
            full(_MODULE_NUM, h),
        ],
        out_specs=pl.BlockSpec((_BLOCK, h), lambda i: (i, 0)),
        out_shape=jax.ShapeDtypeStruct((n, h), jnp.float32),
        compiler_params=pltpu.CompilerParams(
            dimension_semantics=("arbitrary",),
        ),
        interpret=interpret,
    )(mx, gx, wg0, bg0, wg1, bg1, wm, bm)


def _rearrange_gate_out(wg1, bg1):
    gin = wg1.shape[0]
    wp = np.zeros((gin, _MODULE_NUM * _GROUP), dtype=np.float32)
    bp = np.full((1, _MODULE_NUM * _GROUP), _NEG, dtype=np.float32)
    wp = jnp.asarray(wp)
    bp = jnp.asarray(bp)
    off = 0
    for j in range(_MODULE_NUM):
        width = j + 1
        wp = wp.at[:, _GROUP * j:_GROUP * j + width].set(
            wg1[:, off:off + width])
        bp = bp.at[:, _GROUP * j:_GROUP * j + width].set(
            bg1[off:off + width][None, :])
        off += width
    return wp, bp


def kernel(module_input, gate_input, module_Ws, module_bs, gate_Ws, gate_bs,
           interpret=False):
    bf16 = jnp.bfloat16
    mx = module_input.astype(bf16)
    gx = gate_input.astype(bf16)
    wm = jnp.stack(module_Ws).astype(bf16)
    bm = jnp.stack(module_bs)
    wg0 = gate_Ws[0].astype(bf16)
    bg0 = gate_bs[0].reshape(1, -1)
    wg1p, bg1p = _rearrange_gate_out(gate_Ws[1], gate_bs[1])
    return _run(mx, gx, wg0, bg0, wg1p.astype(bf16), bg1p, wm, bm,
                interpret=interpret)


# speedup vs baseline: 1.0150x; 1.0150x over previous
import functools

import numpy as np
import jax
import jax.numpy as jnp
from jax.experimental import pallas as pl
from jax.experimental.pallas import tpu as pltpu

_MODULE_NUM = 8
_HALF = 256
_BLOCK = 2 * _HALF
_GROUP = 8
_NEG = -1e30


def _seg_butterfly(x, combine):
    lanes = x.shape[1]
    lane = jax.lax.broadcasted_iota(jnp.int32, x.shape, 1)
    for k in (1, 2, 4):
        fwd = jnp.roll(x, -k, axis=1)
        bwd = jnp.roll(x, k, axis=1)
        partner = jnp.where((lane & k) == 0, fwd, bwd)
        x = combine(x, partner)
    return x


def _routing_weights64(g64):
    i32 = jnp.int32
    lane = jax.lax.broadcasted_iota(i32, g64.shape, 1)
    m1 = _seg_butterfly(g64, jnp.maximum)
    i1 = _seg_butterfly(jnp.where(g64 >= m1, lane, 64), jnp.minimum)
    first1 = lane == i1
    masked = jnp.where(first1, _NEG, g64)
    m2 = _seg_butterfly(masked, jnp.maximum)
    i2 = _seg_butterfly(jnp.where(masked >= m2, lane, 64), jnp.minimum)
    first2 = lane == i2
    e2 = jnp.exp(m2 - m1)
    w1 = 1.0 / (1.0 + e2)
    zero = jnp.zeros_like(g64)
    return jnp.where(first1, w1, zero) + jnp.where(first2, 1.0 - w1, zero)


def _fused_body(mx_ref, gx_ref, wg0_ref, bg0_ref, wg1_ref, bg1_ref,
                wm_ref, bm_ref, out_ref):
    f32 = jnp.float32
    bf16 = jnp.bfloat16
    g1 = jnp.dot(gx_ref[...], wg0_ref[...], preferred_element_type=f32)
    g1 = jnp.maximum(g1 + bg0_ref[...], 0.0)
    g64 = jnp.dot(g1.astype(bf16), wg1_ref[...],
                  preferred_element_type=f32) + bg1_ref[...]
    wd = _routing_weights64(g64)

    _CH = 128
    rows = [slice(0, _BLOCK)]
    h_dim = wm_ref.shape[2]
    for h in range(1):
        r = rows[h]
        a = jnp.dot(mx_ref[r, :], wm_ref[0], preferred_element_type=f32)
        out = jnp.maximum(a + bm_ref[0:1, :], 0.0)
        outs = [out.astype(bf16)]
        for j in range(1, _MODULE_NUM):
            c0 = _GROUP * (j - 1)
            wcols = [wd[r, c0 + i:c0 + i + 1] for i in range(j)]
            chunks = []
            for s0 in range(0, h_dim, _CH):
                s = slice(s0, s0 + _CH)
                accc = wcols[0] * outs[0][:, s]
                for i in range(1, j):
                    accc = accc + wcols[i] * outs[i][:, s]
                chunks.append(accc)
            fc_in = jnp.concatenate(chunks, axis=1)
            fc = jnp.dot(fc_in.astype(bf16), wm_ref[j],
                         preferred_element_type=f32)
            out = jnp.maximum(fc + bm_ref[j:j + 1, :], 0.0) + fc_in
            outs.append(out.astype(bf16))
        c0 = _GROUP * (_MODULE_NUM - 1)
        wcols = [wd[r, c0 + i:c0 + i + 1] for i in range(_MODULE_NUM)]
        for s0 in range(0, h_dim, _CH):
            s = slice(s0, s0 + _CH)
            accc = wcols[0] * outs[0][:, s]
            for i in range(1, _MODULE_NUM):
                accc = accc + wcols[i] * outs[i][:, s]
            out_ref[r, s] = accc


@functools.partial(jax.jit, static_argnames=("interpret",))
def _run(mx, gx, wg0, bg0, wg1, bg1, wm, bm, interpret=False):
    n, d_in = mx.shape
    h = wm.shape[2]
    gin = gx.shape[1]
    ghid = wg0.shape[1]
    gout = wg1.shape[1]
    grid = (n // _BLOCK,)
    full = lambda *s: pl.BlockSpec(s, lambda i: (0,) * len(s))
    return pl.pallas_call(
        _fused_body,
        grid=grid,
        in_specs=[
            pl.BlockSpec((_BLOCK, d_in), lambda i: (i, 0)),
            pl.BlockSpec((_BLOCK, gin), lambda i: (i, 0)),
            full(gin, ghid),
            full(1, ghid),
            full(ghid, gout),
            full(1, gout),
            full(_MODULE_NUM, d_in, h),
            full(_MODULE_NUM, h),
        ],
        out_specs=pl.BlockSpec((_BLOCK, h), lambda i: (i, 0)),
        out_shape=jax.ShapeDtypeStruct((n, h), jnp.float32),
        compiler_params=pltpu.CompilerParams(
            dimension_semantics=("arbitrary",),
        ),
        interpret=interpret,
    )(mx, gx, wg0, bg0, wg1, bg1, wm, bm)


def _rearrange_gate_out(wg1, bg1):
    gin = wg1.shape[0]
    wp = np.zeros((gin, _MODULE_NUM * _GROUP), dtype=np.float32)
    bp = np.full((1, _MODULE_NUM * _GROUP), _NEG, dtype=np.float32)
    wp = jnp.asarray(wp)
    bp = jnp.asarray(bp)
    off = 0
    for j in range(_MODULE_NUM):
        width = j + 1
        wp = wp.at[:, _GROUP * j:_GROUP * j + width].set(
            wg1[:, off:off + width])
        bp = bp.at[:, _GROUP * j:_GROUP * j + width].set(
            bg1[off:off + width][None, :])
        off += width
    return wp, bp


def kernel(module_input, gate_input, module_Ws, module_bs, gate_Ws, gate_bs,
           interpret=False):
    bf16 = jnp.bfloat16
    mx = module_input.astype(bf16)
    gx = gate_input.astype(bf16)
    wm = jnp.stack(module_Ws).astype(bf16)
    bm = jnp.stack(module_bs)
    wg0 = gate_Ws[0].astype(bf16)
    bg0 = gate_bs[0].reshape(1, -1)
    wg1p, bg1p = _rearrange_gate_out(gate_Ws[1], gate_bs[1])
    return _run(mx, gx, wg0, bg0, wg1p.astype(bf16), bg1p, wm, bm,
                interpret=interpret)


# speedup vs baseline: 1.0624x; 1.0467x over previous
import functools

import numpy as np
import jax
import jax.numpy as jnp
from jax.experimental import pallas as pl
from jax.experimental.pallas import tpu as pltpu

_MODULE_NUM = 8
_HALF = 256
_BLOCK = 2 * _HALF
_GROUP = 8
_NEG = -1e30


def _seg_butterfly(x, combine):
    lanes = x.shape[1]
    lane = jax.lax.broadcasted_iota(jnp.int32, x.shape, 1)
    for k in (1, 2, 4):
        fwd = jnp.roll(x, -k, axis=1)
        bwd = jnp.roll(x, k, axis=1)
        partner = jnp.where((lane & k) == 0, fwd, bwd)
        x = combine(x, partner)
    return x


def _routing_weights64(g64):
    i32 = jnp.int32
    lane = jax.lax.broadcasted_iota(i32, g64.shape, 1)
    m1 = _seg_butterfly(g64, jnp.maximum)
    i1 = _seg_butterfly(jnp.where(g64 >= m1, lane, 64), jnp.minimum)
    first1 = lane == i1
    masked = jnp.where(first1, _NEG, g64)
    m2 = _seg_butterfly(masked, jnp.maximum)
    i2 = _seg_butterfly(jnp.where(masked >= m2, lane, 64), jnp.minimum)
    first2 = lane == i2
    e2 = jnp.exp(m2 - m1)
    w1 = 1.0 / (1.0 + e2)
    zero = jnp.zeros_like(g64)
    return jnp.where(first1, w1, zero) + jnp.where(first2, 1.0 - w1, zero)


def _gate_routing(gx, wg0_ref, bg0_ref, wg1_ref, bg1_ref):
    f32 = jnp.float32
    bf16 = jnp.bfloat16
    g1 = jnp.dot(gx, wg0_ref[...], preferred_element_type=f32)
    g1 = jnp.maximum(g1 + bg0_ref[...], 0.0)
    g64 = jnp.dot(g1.astype(bf16), wg1_ref[...],
                  preferred_element_type=f32) + bg1_ref[...]
    return _routing_weights64(g64)


def _fused_body(mx_ref, gx_ref, gxn_ref, wg0_ref, bg0_ref, wg1_ref, bg1_ref,
                wm_ref, bm_ref, out_ref, wd_ref):
    f32 = jnp.float32
    bf16 = jnp.bfloat16
    step = pl.program_id(0)

    @pl.when(step == 0)
    def _init():
        wd_ref[...] = _gate_routing(gx_ref[...], wg0_ref, bg0_ref,
                                    wg1_ref, bg1_ref)

    wd = wd_ref[...]
    wd_ref[...] = _gate_routing(gxn_ref[...], wg0_ref, bg0_ref,
                                wg1_ref, bg1_ref)

    _CH = 128
    rows = [slice(0, _HALF), slice(_HALF, _BLOCK)]
    h_dim = wm_ref.shape[2]
    for h in range(2):
        r = rows[h]
        a = jnp.dot(mx_ref[r, :], wm_ref[0], preferred_element_type=f32)
        out = jnp.maximum(a + bm_ref[0:1, :], 0.0)
        outs = [out.astype(bf16)]
        for j in range(1, _MODULE_NUM):
            c0 = _GROUP * (j - 1)
            wcols = [wd[r, c0 + i:c0 + i + 1] for i in range(j)]
            chunks = []
            for s0 in range(0, h_dim, _CH):
                s = slice(s0, s0 + _CH)
                accc = wcols[0] * outs[0][:, s]
                for i in range(1, j):
                    accc = accc + wcols[i] * outs[i][:, s]
                chunks.append(accc)
            fc_in = jnp.concatenate(chunks, axis=1)
            fc = jnp.dot(fc_in.astype(bf16), wm_ref[j],
                         preferred_element_type=f32)
            out = jnp.maximum(fc + bm_ref[j:j + 1, :], 0.0) + fc_in
            outs.append(out.astype(bf16))
        c0 = _GROUP * (_MODULE_NUM - 1)
        wcols = [wd[r, c0 + i:c0 + i + 1] for i in range(_MODULE_NUM)]
        for s0 in range(0, h_dim, _CH):
            s = slice(s0, s0 + _CH)
            accc = wcols[0] * outs[0][:, s]
            for i in range(1, _MODULE_NUM):
                accc = accc + wcols[i] * outs[i][:, s]
            out_ref[r, s] = accc


@functools.partial(jax.jit, static_argnames=("interpret",))
def _run(mx, gx, wg0, bg0, wg1, bg1, wm, bm, interpret=False):
    n, d_in = mx.shape
    h = wm.shape[2]
    gin = gx.shape[1]
    ghid = wg0.shape[1]
    gout = wg1.shape[1]
    grid = (n // _BLOCK,)
    last = grid[0] - 1
    full = lambda *s: pl.BlockSpec(s, lambda i: (0,) * len(s))
    return pl.pallas_call(
        _fused_body,
        grid=grid,
        in_specs=[
            pl.BlockSpec((_BLOCK, d_in), lambda i: (i, 0)),
            pl.BlockSpec((_BLOCK, gin), lambda i: (i, 0)),
            pl.BlockSpec((_BLOCK, gin), lambda i: (jnp.minimum(i + 1, last), 0)),
            full(gin, ghid),
            full(1, ghid),
            full(ghid, gout),
            full(1, gout),
            full(_MODULE_NUM, d_in, h),
            full(_MODULE_NUM, h),
        ],
        out_specs=pl.BlockSpec((_BLOCK, h), lambda i: (i, 0)),
        out_shape=jax.ShapeDtypeStruct((n, h), jnp.float32),
        scratch_shapes=[pltpu.VMEM((_BLOCK, _MODULE_NUM * _GROUP),
                                   jnp.float32)],
        compiler_params=pltpu.CompilerParams(
            dimension_semantics=("arbitrary",),
        ),
        interpret=interpret,
    )(mx, gx, gx, wg0, bg0, wg1, bg1, wm, bm)


def _rearrange_gate_out(wg1, bg1):
    gin = wg1.shape[0]
    wp = np.zeros((gin, _MODULE_NUM * _GROUP), dtype=np.float32)
    bp = np.full((1, _MODULE_NUM * _GROUP), _NEG, dtype=np.float32)
    wp = jnp.asarray(wp)
    bp = jnp.asarray(bp)
    off = 0
    for j in range(_MODULE_NUM):
        width = j + 1
        wp = wp.at[:, _GROUP * j:_GROUP * j + width].set(
            wg1[:, off:off + width])
        bp = bp.at[:, _GROUP * j:_GROUP * j + width].set(
            bg1[off:off + width][None, :])
        off += width
    return wp, bp


def kernel(module_input, gate_input, module_Ws, module_bs, gate_Ws, gate_bs,
           interpret=False):
    bf16 = jnp.bfloat16
    mx = module_input.astype(bf16)
    gx = gate_input.astype(bf16)
    wm = jnp.stack(module_Ws).astype(bf16)
    bm = jnp.stack(module_bs)
    wg0 = gate_Ws[0].astype(bf16)
    bg0 = gate_bs[0].reshape(1, -1)
    wg1p, bg1p = _rearrange_gate_out(gate_Ws[1], gate_bs[1])
    return _run(mx, gx, wg0, bg0, wg1p.astype(bf16), bg1p, wm, bm,
                interpret=interpret)
